# baseline (speedup 1.0000x reference)
import functools

import jax
import jax.numpy as jnp
from jax import lax
from jax.experimental import pallas as pl
from jax.experimental.pallas import tpu as pltpu
from jax.experimental.pallas import tpu_sc as plsc

_NUM_E = 8192
_DIM = 256
_N_TOK = 16384
_CC = 0.25

_BM = 256
_NBLK = _N_TOK // _BM

_NC = 2
_NS = 16
_NW = _NC * _NS
_BPW = _N_TOK // _NW
_CH = 128
_NCHUNK = _BPW // _CH

_WINDOWS = ((0, 2736), (2736, 5472), (5472, _NUM_E))


def _dist_kernel(x_ref, x2_ref, e2_ref, e_ref,
                 idx_ref, loss_ref, perp_ref,
                 counts_scr, loss_scr):
    i = pl.program_id(0)

    @pl.when(i == 0)
    def _init():
        counts_scr[...] = jnp.zeros_like(counts_scr)
        loss_scr[0, 0] = 0.0

    x = x_ref[...]
    e = e_ref[...]
    m = lax.dot_general(x.astype(jnp.bfloat16), e.astype(jnp.bfloat16),
                        (((1,), (1,)), ((), ())),
                        preferred_element_type=jnp.float32)
    scores = (x2_ref[...] + e2_ref[...]) - 2.0 * m
    iota_col = lax.broadcasted_iota(jnp.int32, (_BM, _NUM_E), 1)
    acc_v = None
    acc_i = None
    for a, b in _WINDOWS:
        sw = jnp.where((iota_col >= a) & (iota_col < b), scores, jnp.inf)
        i_w = jnp.argmin(sw, axis=1).astype(jnp.int32)
        v_w = jnp.min(sw, axis=1)
        if acc_v is None:
            acc_v = v_w.astype(jnp.bfloat16).astype(jnp.float32)
            acc_i = i_w
        else:
            lt = acc_v < v_w
            keep_i = lt | ((acc_v == v_w) & (acc_i < i_w))
            acc_i = jnp.where(keep_i, acc_i, i_w)
            acc_v = jnp.where(lt, acc_v, v_w).astype(jnp.bfloat16).astype(jnp.float32)
    idx = acc_i
    idx_ref[...] = idx
    chosen = iota_col == idx[:, None]
    loss_scr[0, 0] += jnp.sum(jnp.where(chosen, scores, 0.0))
    counts_scr[...] += jnp.sum(chosen.astype(jnp.float32), axis=0, keepdims=True)

    @pl.when(i == _NBLK - 1)
    def _fin():
        loss_val = _CC * (loss_scr[0, 0] / (_N_TOK * _DIM))
        loss_ref[...] = jnp.reshape(loss_val, (1, 1))
        p = counts_scr[...] / _N_TOK
        perp_ref[...] = jnp.reshape(jnp.exp(-jnp.sum(p * jnp.log(p + 1e-10))), (1, 1))


def _run_dist(flat, x2t, e2c, embeddings):
    return pl.pallas_call(
        _dist_kernel,
        grid=(_NBLK,),
        in_specs=[
            pl.BlockSpec((_BM, _DIM), lambda i: (i, 0)),
            pl.BlockSpec((_BM, 1), lambda i: (i, 0)),
            pl.BlockSpec((1, _NUM_E), lambda i: (0, 0)),
            pl.BlockSpec((_NUM_E, _DIM), lambda i: (0, 0)),
        ],
        out_specs=[
            pl.BlockSpec((_BM,), lambda i: (i,)),
            pl.BlockSpec((1, 1), lambda i: (0, 0)),
            pl.BlockSpec((1, 1), lambda i: (0, 0)),
        ],
        out_shape=[
            jax.ShapeDtypeStruct((_N_TOK,), jnp.int32),
            jax.ShapeDtypeStruct((1, 1), jnp.float32),
            jax.ShapeDtypeStruct((1, 1), jnp.float32),
        ],
        scratch_shapes=[
            pltpu.VMEM((1, _NUM_E), jnp.float32),
            pltpu.SMEM((1, 1), jnp.float32),
        ],
    )(flat, x2t, e2c, embeddings)


def _sc_gather_body(table_hbm, idx_hbm, out_hbm, idx_v, rows_v, sem):
    wid = lax.axis_index("s") * _NC + lax.axis_index("c")
    for k in range(_NCHUNK):
        base = wid * _BPW + k * _CH
        pltpu.sync_copy(idx_hbm.at[pl.ds(base, _CH)], idx_v)
        pltpu.async_copy(table_hbm.at[idx_v], rows_v, sem).wait()
        pltpu.sync_copy(rows_v, out_hbm.at[pl.ds(base, _CH)])


def _run_gather(embeddings, idx):
    mesh = plsc.VectorSubcoreMesh(core_axis_name="c", subcore_axis_name="s")
    k = pl.kernel(
        _sc_gather_body,
        mesh=mesh,
        out_type=jax.ShapeDtypeStruct((_N_TOK, _DIM), jnp.float32),
        scratch_types=[
            pltpu.VMEM((_CH,), jnp.int32),
            pltpu.VMEM((_CH, _DIM), jnp.float32),
            pltpu.SemaphoreType.DMA,
        ],
    )
    return k(embeddings, idx)


def kernel(inputs, embeddings):
    x = jnp.transpose(inputs, (0, 2, 3, 1))
    ishape = x.shape
    flat = x.reshape(-1, _DIM)
    x2t = jnp.sum(flat ** 2, axis=1, keepdims=True)
    e2c = jnp.sum(embeddings ** 2, axis=1)[None, :]
    idx, loss, perp = _run_dist(flat, x2t, e2c, embeddings)
    quant = _run_gather(embeddings, idx)
    quantized_out = jnp.transpose(quant.reshape(ishape), (0, 3, 1, 2))
    loss = loss[0, 0]
    perplexity = perp[0, 0]
    idx_out = idx.reshape(ishape[0], ishape[1], ishape[2])
    return (loss, quantized_out, perplexity, idx_out)
